# baseline (speedup 1.0000x reference)
import functools

import jax
import jax.numpy as jnp
from jax import lax
from jax.experimental import pallas as pl
from jax.experimental.pallas import tpu as pltpu
from jax.experimental.pallas import tpu_sc as plsc

B = 4096
F = 32
NC = 2
NS = 16
NW = NC * NS
BPW = B // NW
CH = BPW // 16


def _mf_body(users_hbm, items_hbm, uemb_hbm, iemb_hbm, ub_hbm, ib_hbm,
             bias_hbm, out_hbm,
             uidx_v, iidx_v, ue_v, ie_v, ub_v, ib_v, bias_v, out_v, sem):
    wid = lax.axis_index("s") * NC + lax.axis_index("c")
    base = wid * BPW

    pltpu.sync_copy(users_hbm.at[pl.ds(base, BPW)], uidx_v)
    pltpu.sync_copy(items_hbm.at[pl.ds(base, BPW)], iidx_v)
    pltpu.sync_copy(bias_hbm, bias_v)

    cp1 = pltpu.async_copy(uemb_hbm.at[uidx_v], ue_v, sem)
    cp2 = pltpu.async_copy(iemb_hbm.at[iidx_v], ie_v, sem)
    cp3 = pltpu.async_copy(ub_hbm.at[uidx_v], ub_v, sem)
    cp4 = pltpu.async_copy(ib_hbm.at[iidx_v], ib_v, sem)
    cp1.wait()
    cp2.wait()
    cp3.wait()
    cp4.wait()

    bias_vec = bias_v[...]
    for c in range(CH):
        rows = c * 16 + lax.iota(jnp.int32, 16)

        def body(f, acc):
            fvec = jnp.full((16,), f, jnp.int32)
            u = plsc.load_gather(ue_v, [rows, fvec])
            v = plsc.load_gather(ie_v, [rows, fvec])
            return acc + u * v

        acc = lax.fori_loop(0, F, body, jnp.zeros((16,), jnp.float32))
        pred = acc + ub_v[pl.ds(c * 16, 16)] + ib_v[pl.ds(c * 16, 16)] + bias_vec
        out_v[pl.ds(c * 16, 16)] = jnp.clip(pred, 1.0, 5.0)

    pltpu.sync_copy(out_v, out_hbm.at[pl.ds(base, BPW)])


def kernel(users, items, user_embeddings, item_embeddings, user_biases,
           item_biases, bias):
    ub_flat = user_biases.reshape(-1)
    ib_flat = item_biases.reshape(-1)
    bias16 = jnp.broadcast_to(bias, (16,))

    mesh = plsc.VectorSubcoreMesh(core_axis_name="c", subcore_axis_name="s")
    run = pl.kernel(
        _mf_body,
        mesh=mesh,
        compiler_params=pltpu.CompilerParams(
            needs_layout_passes=False, use_tc_tiling_on_sc=False),
        out_type=jax.ShapeDtypeStruct((B,), jnp.float32),
        scratch_types=[
            pltpu.VMEM((BPW,), jnp.int32),
            pltpu.VMEM((BPW,), jnp.int32),
            pltpu.VMEM((BPW, F), jnp.float32),
            pltpu.VMEM((BPW, F), jnp.float32),
            pltpu.VMEM((BPW,), jnp.float32),
            pltpu.VMEM((BPW,), jnp.float32),
            pltpu.VMEM((16,), jnp.float32),
            pltpu.VMEM((BPW,), jnp.float32),
            pltpu.SemaphoreType.DMA,
        ],
    )
    return run(users, items, user_embeddings, item_embeddings,
               ub_flat, ib_flat, bias16)


# speedup vs baseline: 3.5972x; 3.5972x over previous
import functools

import jax
import jax.numpy as jnp
from jax import lax
from jax.experimental import pallas as pl
from jax.experimental.pallas import tpu as pltpu
from jax.experimental.pallas import tpu_sc as plsc

B = 4096
F = 32
N = 1000000
TAIL = N - 128
MAIN_LIM = (N // 128) * 128
NC = 2
NS = 16
NW = NC * NS
BPW = B // NW
CH = BPW // 16


def _mf_body(users_hbm, items_hbm, uemb_hbm, iemb_hbm, ub_hbm, ib_hbm,
             ue_tail_hbm, ie_tail_hbm, ub_tail_hbm, ib_tail_hbm, bias_hbm,
             out_hbm,
             uidx_v, iidx_v, ue_tile, ie_tile, ub_tile, ib_tile,
             ue_tail_v, ie_tail_v, ub_tail_v, ib_tail_v,
             ue_cols, ie_cols, ub_cols, ib_cols, bias_v, out_v, sem):
    wid = lax.axis_index("s") * NC + lax.axis_index("c")
    base = wid * BPW

    pltpu.sync_copy(users_hbm.at[pl.ds(base, BPW)], uidx_v)
    pltpu.sync_copy(items_hbm.at[pl.ds(base, BPW)], iidx_v)
    pltpu.sync_copy(bias_hbm, bias_v)
    pltpu.sync_copy(ue_tail_hbm, ue_tail_v)
    pltpu.sync_copy(ie_tail_hbm, ie_tail_v)
    pltpu.sync_copy(ub_tail_hbm, ub_tail_v)
    pltpu.sync_copy(ib_tail_hbm, ib_tail_v)

    lanes = lax.iota(jnp.int32, 16)
    zeros16 = jnp.zeros((16,), jnp.int32)

    def handle(i, idx_v, emb_hbm, b_hbm, emb_tile, b_tile, tail_emb_v,
               tail_b_v, cols_v, bcols_v):
        chunk = idx_v[pl.ds((i // 16) * 16, 16)]
        u = jnp.sum(jnp.where(lanes == (i % 16), chunk, 0))
        is_main = u < MAIN_LIM

        @pl.when(is_main)
        def _():
            start = pl.multiple_of((u // 128) * 128, 128)
            cp1 = pltpu.async_copy(
                emb_hbm.at[:, pl.ds(start, 128)], emb_tile, sem)
            cp2 = pltpu.async_copy(
                b_hbm.at[:, pl.ds(start, 128)], b_tile, sem)
            cp1.wait()
            cp2.wait()
            col = jnp.broadcast_to(u % 128, (16,)).astype(jnp.int32)
            e16 = jnp.broadcast_to(i, (16,)).astype(jnp.int32)
            lo = plsc.load_gather(emb_tile, [lanes, col])
            hi = plsc.load_gather(emb_tile, [lanes + 16, col])
            plsc.store_scatter(cols_v, [lanes, e16], lo)
            plsc.store_scatter(cols_v, [lanes + 16, e16], hi)
            bv = plsc.load_gather(b_tile, [zeros16, col])
            plsc.store_scatter(bcols_v, [e16], bv, mask=lanes == 0)

        @pl.when(jnp.logical_not(is_main))
        def _():
            col = jnp.broadcast_to(u - TAIL, (16,)).astype(jnp.int32)
            e16 = jnp.broadcast_to(i, (16,)).astype(jnp.int32)
            lo = plsc.load_gather(tail_emb_v, [lanes, col])
            hi = plsc.load_gather(tail_emb_v, [lanes + 16, col])
            plsc.store_scatter(cols_v, [lanes, e16], lo)
            plsc.store_scatter(cols_v, [lanes + 16, e16], hi)
            bv = plsc.load_gather(tail_b_v, [zeros16, col])
            plsc.store_scatter(bcols_v, [e16], bv, mask=lanes == 0)

    def body(i, carry):
        handle(i, uidx_v, uemb_hbm, ub_hbm, ue_tile, ub_tile,
               ue_tail_v, ub_tail_v, ue_cols, ub_cols)
        handle(i, iidx_v, iemb_hbm, ib_hbm, ie_tile, ib_tile,
               ie_tail_v, ib_tail_v, ie_cols, ib_cols)
        return carry

    lax.fori_loop(0, BPW, body, 0)

    bias_vec = bias_v[...]
    for c in range(CH):
        sl = pl.ds(c * 16, 16)
        acc = ub_cols[sl] + ib_cols[sl] + bias_vec
        for f in range(F):
            acc = acc + ue_cols[f, sl] * ie_cols[f, sl]
        out_v[sl] = jnp.clip(acc, 1.0, 5.0)

    pltpu.sync_copy(out_v, out_hbm.at[pl.ds(base, BPW)])


def kernel(users, items, user_embeddings, item_embeddings, user_biases,
           item_biases, bias):
    uemb_t = user_embeddings.T
    iemb_t = item_embeddings.T
    ub_t = user_biases.T
    ib_t = item_biases.T
    ue_tail = uemb_t[:, TAIL:]
    ie_tail = iemb_t[:, TAIL:]
    ub_tail = ub_t[:, TAIL:]
    ib_tail = ib_t[:, TAIL:]
    bias16 = jnp.broadcast_to(bias, (16,))

    mesh = plsc.VectorSubcoreMesh(core_axis_name="c", subcore_axis_name="s")
    run = pl.kernel(
        _mf_body,
        mesh=mesh,
        compiler_params=pltpu.CompilerParams(needs_layout_passes=False),
        out_type=jax.ShapeDtypeStruct((B,), jnp.float32),
        scratch_types=[
            pltpu.VMEM((BPW,), jnp.int32),
            pltpu.VMEM((BPW,), jnp.int32),
            pltpu.VMEM((F, 128), jnp.float32),
            pltpu.VMEM((F, 128), jnp.float32),
            pltpu.VMEM((1, 128), jnp.float32),
            pltpu.VMEM((1, 128), jnp.float32),
            pltpu.VMEM((F, 128), jnp.float32),
            pltpu.VMEM((F, 128), jnp.float32),
            pltpu.VMEM((1, 128), jnp.float32),
            pltpu.VMEM((1, 128), jnp.float32),
            pltpu.VMEM((F, BPW), jnp.float32),
            pltpu.VMEM((F, BPW), jnp.float32),
            pltpu.VMEM((BPW,), jnp.float32),
            pltpu.VMEM((BPW,), jnp.float32),
            pltpu.VMEM((16,), jnp.float32),
            pltpu.VMEM((BPW,), jnp.float32),
            pltpu.SemaphoreType.DMA,
        ],
    )
    return run(users, items, uemb_t, iemb_t, ub_t, ib_t,
               ue_tail, ie_tail, ub_tail, ib_tail, bias16)


# speedup vs baseline: 10.6997x; 2.9744x over previous
import functools

import jax
import jax.numpy as jnp
from jax import lax
from jax.experimental import pallas as pl
from jax.experimental.pallas import tpu as pltpu
from jax.experimental.pallas import tpu_sc as plsc

B = 4096
F = 32
N = 1000000
TAIL = N - 128
LAST_TILE = (N - 128) // 128 * 128
RING = 8
NC = 2
NS = 16
NW = NC * NS
BPW = B // NW
CH = BPW // 16


def _mf_body(users_hbm, items_hbm, uemb_hbm, iemb_hbm, ub_hbm, ib_hbm,
             ue_tail_hbm, ie_tail_hbm, ub_tail_hbm, ib_tail_hbm, bias_hbm,
             out_hbm,
             uidx_v, iidx_v, ue_tiles, ie_tiles, ub_tiles, ib_tiles,
             ue_tail_v, ie_tail_v, ub_tail_v, ib_tail_v,
             ue_cols, ie_cols, ub_cols, ib_cols, ucol_buf, icol_buf,
             bias_v, out_v, sems):
    wid = lax.axis_index("s") * NC + lax.axis_index("c")
    base = wid * BPW

    pltpu.sync_copy(users_hbm.at[pl.ds(base, BPW)], uidx_v)
    pltpu.sync_copy(items_hbm.at[pl.ds(base, BPW)], iidx_v)
    pltpu.sync_copy(bias_hbm, bias_v)
    pltpu.sync_copy(ue_tail_hbm, ue_tail_v)
    pltpu.sync_copy(ie_tail_hbm, ie_tail_v)
    pltpu.sync_copy(ub_tail_hbm, ub_tail_v)
    pltpu.sync_copy(ib_tail_hbm, ib_tail_v)

    lanes = lax.iota(jnp.int32, 16)
    zeros16 = jnp.zeros((16,), jnp.int32)
    lane0 = lanes == 0

    def extract_scalar(idx_v, j):
        chunk = idx_v[pl.ds((j // 16) * 16, 16)]
        return jnp.sum(jnp.where(lanes == (j % 16), chunk, 0))

    def fire(j, r):
        u = extract_scalar(uidx_v, j)
        t = extract_scalar(iidx_v, j)
        su = pl.multiple_of(jnp.minimum((u // 128) * 128, LAST_TILE), 128)
        st = pl.multiple_of(jnp.minimum((t // 128) * 128, LAST_TILE), 128)
        pltpu.async_copy(uemb_hbm.at[:, pl.ds(su, 128)], ue_tiles.at[r],
                         sems.at[r])
        pltpu.async_copy(ub_hbm.at[:, pl.ds(su, 128)], ub_tiles.at[r],
                         sems.at[r])
        pltpu.async_copy(iemb_hbm.at[:, pl.ds(st, 128)], ie_tiles.at[r],
                         sems.at[r])
        pltpu.async_copy(ib_hbm.at[:, pl.ds(st, 128)], ib_tiles.at[r],
                         sems.at[r])
        j16 = jnp.broadcast_to(j, (16,)).astype(jnp.int32)
        plsc.store_scatter(ucol_buf, [j16],
                           jnp.broadcast_to(u - su, (16,)).astype(jnp.int32),
                           mask=lane0)
        plsc.store_scatter(icol_buf, [j16],
                           jnp.broadcast_to(t - st, (16,)).astype(jnp.int32),
                           mask=lane0)

    def drain(r):
        pltpu.make_async_copy(uemb_hbm.at[:, pl.ds(0, 128)],
                              ue_tiles.at[r], sems.at[r]).wait()
        pltpu.make_async_copy(ub_hbm.at[:, pl.ds(0, 128)],
                              ub_tiles.at[r], sems.at[r]).wait()
        pltpu.make_async_copy(iemb_hbm.at[:, pl.ds(0, 128)],
                              ie_tiles.at[r], sems.at[r]).wait()
        pltpu.make_async_copy(ib_hbm.at[:, pl.ds(0, 128)],
                              ib_tiles.at[r], sems.at[r]).wait()

    def extract(j, r, col_buf, tiles, b_tiles, tail_v, tail_b_v, cols_v,
                bcols_v):
        j16 = jnp.broadcast_to(j, (16,)).astype(jnp.int32)
        col = plsc.load_gather(col_buf, [j16])
        sel = col < 128
        cm = jnp.minimum(col, 127)
        ct = jnp.clip(col - 64, 0, 127)
        lo = plsc.load_gather(tiles.at[r], [lanes, cm])
        hi = plsc.load_gather(tiles.at[r], [lanes + 16, cm])
        lo_t = plsc.load_gather(tail_v, [lanes, ct])
        hi_t = plsc.load_gather(tail_v, [lanes + 16, ct])
        plsc.store_scatter(cols_v, [lanes, j16], jnp.where(sel, lo, lo_t))
        plsc.store_scatter(cols_v, [lanes + 16, j16], jnp.where(sel, hi, hi_t))
        bv = plsc.load_gather(b_tiles.at[r], [zeros16, cm])
        bv_t = plsc.load_gather(tail_b_v, [zeros16, ct])
        plsc.store_scatter(bcols_v, [j16], jnp.where(sel, bv, bv_t),
                           mask=lane0)

    for r in range(RING):
        fire(r, r)

    def body(g, carry):
        for r in range(RING):
            j = g * RING + r
            drain(r)
            extract(j, r, ucol_buf, ue_tiles, ub_tiles, ue_tail_v, ub_tail_v,
                    ue_cols, ub_cols)
            extract(j, r, icol_buf, ie_tiles, ib_tiles, ie_tail_v, ib_tail_v,
                    ie_cols, ib_cols)
            jn = j + RING

            @pl.when(jn < BPW)
            def _():
                fire(jn, r)
        return carry

    lax.fori_loop(0, BPW // RING, body, 0)

    bias_vec = bias_v[...]
    for c in range(CH):
        sl = pl.ds(c * 16, 16)
        acc = ub_cols[sl] + ib_cols[sl] + bias_vec
        for f in range(F):
            acc = acc + ue_cols[f, sl] * ie_cols[f, sl]
        out_v[sl] = jnp.clip(acc, 1.0, 5.0)

    pltpu.sync_copy(out_v, out_hbm.at[pl.ds(base, BPW)])


def kernel(users, items, user_embeddings, item_embeddings, user_biases,
           item_biases, bias):
    uemb_t = user_embeddings.T
    iemb_t = item_embeddings.T
    ub_t = user_biases.T
    ib_t = item_biases.T
    ue_tail = uemb_t[:, TAIL:]
    ie_tail = iemb_t[:, TAIL:]
    ub_tail = ub_t[:, TAIL:]
    ib_tail = ib_t[:, TAIL:]
    bias16 = jnp.broadcast_to(bias, (16,))

    mesh = plsc.VectorSubcoreMesh(core_axis_name="c", subcore_axis_name="s")
    run = pl.kernel(
        _mf_body,
        mesh=mesh,
        compiler_params=pltpu.CompilerParams(needs_layout_passes=False),
        out_type=jax.ShapeDtypeStruct((B,), jnp.float32),
        scratch_types=[
            pltpu.VMEM((BPW,), jnp.int32),
            pltpu.VMEM((BPW,), jnp.int32),
            pltpu.VMEM((RING, F, 128), jnp.float32),
            pltpu.VMEM((RING, F, 128), jnp.float32),
            pltpu.VMEM((RING, 1, 128), jnp.float32),
            pltpu.VMEM((RING, 1, 128), jnp.float32),
            pltpu.VMEM((F, 128), jnp.float32),
            pltpu.VMEM((F, 128), jnp.float32),
            pltpu.VMEM((1, 128), jnp.float32),
            pltpu.VMEM((1, 128), jnp.float32),
            pltpu.VMEM((F, BPW), jnp.float32),
            pltpu.VMEM((F, BPW), jnp.float32),
            pltpu.VMEM((BPW,), jnp.float32),
            pltpu.VMEM((BPW,), jnp.float32),
            pltpu.VMEM((BPW,), jnp.int32),
            pltpu.VMEM((BPW,), jnp.int32),
            pltpu.VMEM((16,), jnp.float32),
            pltpu.VMEM((BPW,), jnp.float32),
            pltpu.SemaphoreType.DMA((RING,)),
        ],
    )
    return run(users, items, uemb_t, iemb_t, ub_t, ib_t,
               ue_tail, ie_tail, ub_tail, ib_tail, bias16)


# speedup vs baseline: 10.9538x; 1.0237x over previous
import functools

import jax
import jax.numpy as jnp
from jax import lax
from jax.experimental import pallas as pl
from jax.experimental.pallas import tpu as pltpu
from jax.experimental.pallas import tpu_sc as plsc

B = 4096
F = 32
N = 1000000
TAIL64 = (N // 128) * 128
LAST_TILE = (N - 128) // 128 * 128
RING = 8
NC = 2
NS = 16
NW = NC * NS
BPW = B // NW
CH = BPW // 16


def _mf_body(users_hbm, items_hbm, uemb_hbm, iemb_hbm, ub_hbm, ib_hbm,
             bias_hbm, out_hbm,
             uidx_v, iidx_v, ue_tiles, ie_tiles, ub_tiles, ib_tiles,
             ue_tail_v, ie_tail_v, ub_tail_v, ib_tail_v,
             ue_cols, ie_cols, ub_cols, ib_cols, ucol_buf, icol_buf,
             bias_v, out_v, sems):
    wid = lax.axis_index("s") * NC + lax.axis_index("c")
    base = wid * BPW

    pltpu.sync_copy(users_hbm.at[pl.ds(base, BPW)], uidx_v)
    pltpu.sync_copy(items_hbm.at[pl.ds(base, BPW)], iidx_v)
    pltpu.sync_copy(bias_hbm, bias_v)
    pltpu.sync_copy(uemb_hbm.at[:, pl.ds(TAIL64, 64)], ue_tail_v)
    pltpu.sync_copy(iemb_hbm.at[:, pl.ds(TAIL64, 64)], ie_tail_v)
    pltpu.sync_copy(ub_hbm.at[:, pl.ds(TAIL64, 64)], ub_tail_v)
    pltpu.sync_copy(ib_hbm.at[:, pl.ds(TAIL64, 64)], ib_tail_v)

    lanes = lax.iota(jnp.int32, 16)
    zeros16 = jnp.zeros((16,), jnp.int32)
    lane0 = lanes == 0

    def extract_scalar(idx_v, j):
        chunk = idx_v[pl.ds((j // 16) * 16, 16)]
        return jnp.sum(jnp.where(lanes == (j % 16), chunk, 0))

    def fire(j, r):
        u = extract_scalar(uidx_v, j)
        t = extract_scalar(iidx_v, j)
        su = pl.multiple_of(jnp.minimum((u // 128) * 128, LAST_TILE), 128)
        st = pl.multiple_of(jnp.minimum((t // 128) * 128, LAST_TILE), 128)
        pltpu.async_copy(uemb_hbm.at[:, pl.ds(su, 128)], ue_tiles.at[r],
                         sems.at[r])
        pltpu.async_copy(ub_hbm.at[:, pl.ds(su, 128)], ub_tiles.at[r],
                         sems.at[r])
        pltpu.async_copy(iemb_hbm.at[:, pl.ds(st, 128)], ie_tiles.at[r],
                         sems.at[r])
        pltpu.async_copy(ib_hbm.at[:, pl.ds(st, 128)], ib_tiles.at[r],
                         sems.at[r])
        j16 = jnp.broadcast_to(j, (16,)).astype(jnp.int32)
        plsc.store_scatter(ucol_buf, [j16],
                           jnp.broadcast_to(u - su, (16,)).astype(jnp.int32),
                           mask=lane0)
        plsc.store_scatter(icol_buf, [j16],
                           jnp.broadcast_to(t - st, (16,)).astype(jnp.int32),
                           mask=lane0)

    def drain(r):
        pltpu.make_async_copy(uemb_hbm.at[:, pl.ds(0, 128)],
                              ue_tiles.at[r], sems.at[r]).wait()
        pltpu.make_async_copy(ub_hbm.at[:, pl.ds(0, 128)],
                              ub_tiles.at[r], sems.at[r]).wait()
        pltpu.make_async_copy(iemb_hbm.at[:, pl.ds(0, 128)],
                              ie_tiles.at[r], sems.at[r]).wait()
        pltpu.make_async_copy(ib_hbm.at[:, pl.ds(0, 128)],
                              ib_tiles.at[r], sems.at[r]).wait()

    def extract(j, r, col_buf, tiles, b_tiles, tail_v, tail_b_v, cols_v,
                bcols_v):
        j16 = jnp.broadcast_to(j, (16,)).astype(jnp.int32)
        col = plsc.load_gather(col_buf, [j16])
        sel = col < 128
        cm = jnp.minimum(col, 127)
        ct = jnp.clip(col - 128, 0, 63)
        lo = plsc.load_gather(tiles.at[r], [lanes, cm])
        hi = plsc.load_gather(tiles.at[r], [lanes + 16, cm])
        lo_t = plsc.load_gather(tail_v, [lanes, ct])
        hi_t = plsc.load_gather(tail_v, [lanes + 16, ct])
        plsc.store_scatter(cols_v, [lanes, j16], jnp.where(sel, lo, lo_t))
        plsc.store_scatter(cols_v, [lanes + 16, j16], jnp.where(sel, hi, hi_t))
        bv = plsc.load_gather(b_tiles.at[r], [zeros16, cm])
        bv_t = plsc.load_gather(tail_b_v, [zeros16, ct])
        plsc.store_scatter(bcols_v, [j16], jnp.where(sel, bv, bv_t),
                           mask=lane0)

    for r in range(RING):
        fire(r, r)

    def body(g, carry):
        for r in range(RING):
            j = g * RING + r
            drain(r)
            extract(j, r, ucol_buf, ue_tiles, ub_tiles, ue_tail_v, ub_tail_v,
                    ue_cols, ub_cols)
            extract(j, r, icol_buf, ie_tiles, ib_tiles, ie_tail_v, ib_tail_v,
                    ie_cols, ib_cols)
            jn = j + RING

            @pl.when(jn < BPW)
            def _():
                fire(jn, r)
        return carry

    lax.fori_loop(0, BPW // RING, body, 0)

    bias_vec = bias_v[...]
    for c in range(CH):
        sl = pl.ds(c * 16, 16)
        acc = ub_cols[sl] + ib_cols[sl] + bias_vec
        for f in range(F):
            acc = acc + ue_cols[f, sl] * ie_cols[f, sl]
        out_v[sl] = jnp.clip(acc, 1.0, 5.0)

    pltpu.sync_copy(out_v, out_hbm.at[pl.ds(base, BPW)])


def kernel(users, items, user_embeddings, item_embeddings, user_biases,
           item_biases, bias):
    uemb_t = user_embeddings.T
    iemb_t = item_embeddings.T
    ub_t = user_biases.T
    ib_t = item_biases.T
    bias16 = jnp.broadcast_to(bias, (16,))

    mesh = plsc.VectorSubcoreMesh(core_axis_name="c", subcore_axis_name="s")
    run = pl.kernel(
        _mf_body,
        mesh=mesh,
        compiler_params=pltpu.CompilerParams(needs_layout_passes=False),
        out_type=jax.ShapeDtypeStruct((B,), jnp.float32),
        scratch_types=[
            pltpu.VMEM((BPW,), jnp.int32),
            pltpu.VMEM((BPW,), jnp.int32),
            pltpu.VMEM((RING, F, 128), jnp.float32),
            pltpu.VMEM((RING, F, 128), jnp.float32),
            pltpu.VMEM((RING, 1, 128), jnp.float32),
            pltpu.VMEM((RING, 1, 128), jnp.float32),
            pltpu.VMEM((F, 64), jnp.float32),
            pltpu.VMEM((F, 64), jnp.float32),
            pltpu.VMEM((1, 64), jnp.float32),
            pltpu.VMEM((1, 64), jnp.float32),
            pltpu.VMEM((F, BPW), jnp.float32),
            pltpu.VMEM((F, BPW), jnp.float32),
            pltpu.VMEM((BPW,), jnp.float32),
            pltpu.VMEM((BPW,), jnp.float32),
            pltpu.VMEM((BPW,), jnp.int32),
            pltpu.VMEM((BPW,), jnp.int32),
            pltpu.VMEM((16,), jnp.float32),
            pltpu.VMEM((BPW,), jnp.float32),
            pltpu.SemaphoreType.DMA((RING,)),
        ],
    )
    return run(users, items, uemb_t, iemb_t, ub_t, ib_t, bias16)


# speedup vs baseline: 10.9988x; 1.0041x over previous
import functools

import jax
import jax.numpy as jnp
from jax import lax
from jax.experimental import pallas as pl
from jax.experimental.pallas import tpu as pltpu
from jax.experimental.pallas import tpu_sc as plsc

B = 4096
F = 32
N = 1000000
TAIL64 = (N // 128) * 128
LAST_TILE = (N - 128) // 128 * 128
RING = 8
NC = 2
NS = 16
NW = NC * NS
BPW = B // NW
CH = BPW // 16


def _mf_body(users_hbm, items_hbm, uemb_hbm, iemb_hbm, ub_hbm, ib_hbm,
             bias_hbm, out_hbm,
             uidx_v, iidx_v, ue_tiles, ie_tiles, ub_tiles, ib_tiles,
             ue_tail_v, ie_tail_v, ub_tail_v, ib_tail_v,
             ue_cols, ie_cols, ub_cols, ib_cols, ucol_buf, icol_buf,
             bias_v, out_v, sems):
    wid = lax.axis_index("s") * NC + lax.axis_index("c")
    base = wid * BPW

    pltpu.sync_copy(users_hbm.at[pl.ds(base, BPW)], uidx_v)
    pltpu.sync_copy(items_hbm.at[pl.ds(base, BPW)], iidx_v)
    pltpu.sync_copy(bias_hbm, bias_v.at[pl.ds(0, 1)])
    pltpu.sync_copy(uemb_hbm.at[:, pl.ds(TAIL64, 64)], ue_tail_v)
    pltpu.sync_copy(iemb_hbm.at[:, pl.ds(TAIL64, 64)], ie_tail_v)
    pltpu.sync_copy(ub_hbm.at[:, pl.ds(TAIL64, 64)], ub_tail_v)
    pltpu.sync_copy(ib_hbm.at[:, pl.ds(TAIL64, 64)], ib_tail_v)

    lanes = lax.iota(jnp.int32, 16)
    zeros16 = jnp.zeros((16,), jnp.int32)
    lane0 = lanes == 0

    def extract_scalar(idx_v, j):
        chunk = idx_v[pl.ds((j // 16) * 16, 16)]
        return jnp.sum(jnp.where(lanes == (j % 16), chunk, 0))

    def fire(j, r):
        u = extract_scalar(uidx_v, j)
        t = extract_scalar(iidx_v, j)
        su = pl.multiple_of(jnp.minimum((u // 128) * 128, LAST_TILE), 128)
        st = pl.multiple_of(jnp.minimum((t // 128) * 128, LAST_TILE), 128)
        pltpu.async_copy(uemb_hbm.at[:, pl.ds(su, 128)], ue_tiles.at[r],
                         sems.at[r])
        pltpu.async_copy(ub_hbm.at[:, pl.ds(su, 128)], ub_tiles.at[r],
                         sems.at[r])
        pltpu.async_copy(iemb_hbm.at[:, pl.ds(st, 128)], ie_tiles.at[r],
                         sems.at[r])
        pltpu.async_copy(ib_hbm.at[:, pl.ds(st, 128)], ib_tiles.at[r],
                         sems.at[r])
        j16 = jnp.broadcast_to(j, (16,)).astype(jnp.int32)
        plsc.store_scatter(ucol_buf, [j16],
                           jnp.broadcast_to(u - su, (16,)).astype(jnp.int32),
                           mask=lane0)
        plsc.store_scatter(icol_buf, [j16],
                           jnp.broadcast_to(t - st, (16,)).astype(jnp.int32),
                           mask=lane0)

    def drain(r):
        pltpu.make_async_copy(uemb_hbm.at[:, pl.ds(0, 128)],
                              ue_tiles.at[r], sems.at[r]).wait()
        pltpu.make_async_copy(ub_hbm.at[:, pl.ds(0, 128)],
                              ub_tiles.at[r], sems.at[r]).wait()
        pltpu.make_async_copy(iemb_hbm.at[:, pl.ds(0, 128)],
                              ie_tiles.at[r], sems.at[r]).wait()
        pltpu.make_async_copy(ib_hbm.at[:, pl.ds(0, 128)],
                              ib_tiles.at[r], sems.at[r]).wait()

    def extract(j, r, col_buf, tiles, b_tiles, tail_v, tail_b_v, cols_v,
                bcols_v):
        j16 = jnp.broadcast_to(j, (16,)).astype(jnp.int32)
        col = plsc.load_gather(col_buf, [j16])
        sel = col < 128
        cm = jnp.minimum(col, 127)
        ct = jnp.clip(col - 128, 0, 63)
        lo = plsc.load_gather(tiles.at[r], [lanes, cm])
        hi = plsc.load_gather(tiles.at[r], [lanes + 16, cm])
        lo_t = plsc.load_gather(tail_v, [lanes, ct])
        hi_t = plsc.load_gather(tail_v, [lanes + 16, ct])
        plsc.store_scatter(cols_v, [lanes, j16], jnp.where(sel, lo, lo_t))
        plsc.store_scatter(cols_v, [lanes + 16, j16], jnp.where(sel, hi, hi_t))
        bv = plsc.load_gather(b_tiles.at[r], [zeros16, cm])
        bv_t = plsc.load_gather(tail_b_v, [zeros16, ct])
        plsc.store_scatter(bcols_v, [j16], jnp.where(sel, bv, bv_t),
                           mask=lane0)

    for r in range(RING):
        fire(r, r)

    def body(g, carry):
        for r in range(RING):
            j = g * RING + r
            drain(r)
            extract(j, r, ucol_buf, ue_tiles, ub_tiles, ue_tail_v, ub_tail_v,
                    ue_cols, ub_cols)
            extract(j, r, icol_buf, ie_tiles, ib_tiles, ie_tail_v, ib_tail_v,
                    ie_cols, ib_cols)
            jn = j + RING

            @pl.when(jn < BPW)
            def _():
                fire(jn, r)
        return carry

    lax.fori_loop(0, BPW // RING, body, 0)

    bias_s = jnp.sum(jnp.where(lane0, bias_v[...], 0.0))
    bias_vec = jnp.broadcast_to(bias_s, (16,))
    for c in range(CH):
        sl = pl.ds(c * 16, 16)
        acc = ub_cols[sl] + ib_cols[sl] + bias_vec
        for f in range(F):
            acc = acc + ue_cols[f, sl] * ie_cols[f, sl]
        out_v[sl] = jnp.clip(acc, 1.0, 5.0)

    pltpu.sync_copy(out_v, out_hbm.at[pl.ds(base, BPW)])


def kernel(users, items, user_embeddings, item_embeddings, user_biases,
           item_biases, bias):
    uemb_t = user_embeddings.T
    iemb_t = item_embeddings.T
    ub_t = user_biases.T
    ib_t = item_biases.T
    mesh = plsc.VectorSubcoreMesh(core_axis_name="c", subcore_axis_name="s")
    run = pl.kernel(
        _mf_body,
        mesh=mesh,
        compiler_params=pltpu.CompilerParams(needs_layout_passes=False),
        out_type=jax.ShapeDtypeStruct((B,), jnp.float32),
        scratch_types=[
            pltpu.VMEM((BPW,), jnp.int32),
            pltpu.VMEM((BPW,), jnp.int32),
            pltpu.VMEM((RING, F, 128), jnp.float32),
            pltpu.VMEM((RING, F, 128), jnp.float32),
            pltpu.VMEM((RING, 1, 128), jnp.float32),
            pltpu.VMEM((RING, 1, 128), jnp.float32),
            pltpu.VMEM((F, 64), jnp.float32),
            pltpu.VMEM((F, 64), jnp.float32),
            pltpu.VMEM((1, 64), jnp.float32),
            pltpu.VMEM((1, 64), jnp.float32),
            pltpu.VMEM((F, BPW), jnp.float32),
            pltpu.VMEM((F, BPW), jnp.float32),
            pltpu.VMEM((BPW,), jnp.float32),
            pltpu.VMEM((BPW,), jnp.float32),
            pltpu.VMEM((BPW,), jnp.int32),
            pltpu.VMEM((BPW,), jnp.int32),
            pltpu.VMEM((16,), jnp.float32),
            pltpu.VMEM((BPW,), jnp.float32),
            pltpu.SemaphoreType.DMA((RING,)),
        ],
    )
    return run(users, items, uemb_t, iemb_t, ub_t, ib_t, bias)
